# Initial kernel scaffold; baseline (speedup 1.0000x reference)
import jax
import jax.numpy as jnp
from jax.experimental import pallas as pl


def kernel(x, edge_index, gamma1, beta1, W1, b1, gamma2, beta2, W2, b2):
    raise NotImplementedError("write your pallas kernel here")


# speedup vs baseline: 12.1945x; 12.1945x over previous
import functools

import jax
import jax.numpy as jnp
from jax import lax
from jax.experimental import pallas as pl
from jax.experimental.pallas import tpu as pltpu
from jax.experimental.pallas import tpu_sc as plsc

_N = 10000
_C = 128
_EPS = 1e-5

_NC = 2
_NS = 16
_NW = _NC * _NS
_CHUNK = 128
_N_ACC = 10240
_RPT = _N_ACC // _NS
_ZR = 128


def _f32(shape):
    return jax.ShapeDtypeStruct(shape, jnp.float32)


def _deg_body(nch, dst_hbm, out_hbm, idx_d, obuf, zbuf, acc):
    cid = lax.axis_index("c")
    sid = lax.axis_index("s")
    wid = cid * _NS + sid
    one16 = jnp.ones((16,), jnp.float32)
    zero16 = jnp.zeros((16,), jnp.float32)

    def fill(i, carry):
        zbuf[pl.ds(i * 16, 16)] = zero16
        return carry
    lax.fori_loop(0, _RPT // 16, fill, 0)

    def fill1(i, carry):
        obuf[pl.ds(i * 16, 16)] = one16
        return carry
    lax.fori_loop(0, _CHUNK // 16, fill1, 0)

    pltpu.sync_copy(zbuf, acc.at[pl.ds(sid * _RPT, _RPT)])
    plsc.subcore_barrier()
    pltpu.sync_copy(dst_hbm.at[wid], idx_d)

    def body(j, carry):
        pltpu.sync_copy(obuf, acc.at[idx_d.at[j]], add=True)
        return carry
    lax.fori_loop(0, nch, body, 0)
    plsc.subcore_barrier()
    pltpu.sync_copy(acc.at[pl.ds(sid * _RPT, _RPT)],
                    out_hbm.at[cid, pl.ds(sid * _RPT, _RPT)])


def _agg_body(nch, z_hbm, src_hbm, dst_hbm, out_hbm,
              idx_s, idx_d, rows, acc, gs0):
    cid = lax.axis_index("c")
    sid = lax.axis_index("s")
    wid = cid * _NS + sid
    zero16 = jnp.zeros((16,), jnp.float32)
    lanes = _C // 16

    def fill(i, carry):
        r = i // lanes
        c0 = (i % lanes) * 16
        rows[r, pl.ds(c0, 16)] = zero16
        return carry
    lax.fori_loop(0, _CHUNK * lanes, fill, 0)

    for k in range(_RPT // _CHUNK):
        pltpu.sync_copy(rows, acc.at[pl.ds(sid * _RPT + k * _CHUNK, _CHUNK)])
    plsc.subcore_barrier()

    pltpu.sync_copy(src_hbm.at[wid], idx_s)
    pltpu.sync_copy(dst_hbm.at[wid], idx_d)

    def body(j, carry):
        pltpu.async_copy(z_hbm.at[idx_s.at[j]], rows, gs0).wait()
        pltpu.sync_copy(rows, acc.at[idx_d.at[j]], add=True)
        return carry
    lax.fori_loop(0, nch, body, 0)
    plsc.subcore_barrier()
    pltpu.sync_copy(acc.at[pl.ds(sid * _RPT, _RPT)],
                    out_hbm.at[cid, pl.ds(sid * _RPT, _RPT)])


@functools.lru_cache(maxsize=None)
def _make_sc_kernels(nch):
    mesh = plsc.VectorSubcoreMesh(core_axis_name="c", subcore_axis_name="s")
    deg = pl.kernel(
        functools.partial(_deg_body, nch),
        out_type=_f32((_NC, _N_ACC)),
        mesh=mesh,
        scratch_types=[
            pltpu.VMEM((nch, _CHUNK), jnp.int32),
            pltpu.VMEM((_CHUNK,), jnp.float32),
            pltpu.VMEM((_RPT,), jnp.float32),
            pltpu.VMEM_SHARED((_N_ACC,), jnp.float32),
        ],
    )
    agg = pl.kernel(
        functools.partial(_agg_body, nch),
        out_type=_f32((_NC, _N_ACC, _C)),
        mesh=mesh,
        scratch_types=[
            pltpu.VMEM((nch, _CHUNK), jnp.int32),
            pltpu.VMEM((nch, _CHUNK), jnp.int32),
            pltpu.VMEM((_CHUNK, _C), jnp.float32),
            pltpu.VMEM_SHARED((_N_ACC, _C), jnp.float32),
            pltpu.SemaphoreType.DMA,
        ],
    )
    return deg, agg


def _bn(x, gamma, beta):
    mean = jnp.mean(x, axis=0, keepdims=True)
    var = jnp.mean((x - mean) ** 2, axis=0, keepdims=True)
    return (x - mean) * lax.rsqrt(var + _EPS) * gamma + beta


def _dot(a, b):
    return jnp.dot(a, b, preferred_element_type=jnp.float32)


def _tc1_body(x, d0, d1, g1, b1, xb_o, z1_o, dinv_o):
    deg = d0[...] + d1[...]
    dinv = jnp.where(deg > 0.0, lax.rsqrt(jnp.maximum(deg, 1.0)), 0.0)
    xb = _bn(x[...], g1[...], b1[...])
    xb_o[...] = xb
    z1_o[...] = xb * dinv
    dinv_o[...] = dinv


def _tc3_body(p0, p1, dinv, xin, w0, w1, acc_o, z_o):
    dv = dinv[...]
    tx1 = -dv * (p0[...] + p1[...])
    acc_o[...] = _dot(xin[...], w0[...]) + _dot(tx1, w1[...])
    z_o[...] = dv * tx1


def _tc5_body(p0, p1, dinv, xb, acc1, w2, b1, g2, be2, hb_o, z3_o):
    dv = dinv[...]
    tx2 = -2.0 * dv * (p0[...] + p1[...]) - xb[...]
    h = jax.nn.relu(acc1[...] + _dot(tx2, w2[...]) + b1[...])
    hb = _bn(h, g2[...], be2[...])
    hb_o[...] = hb
    z3_o[...] = dv * hb


def _tc9_body(p0, p1, dinv, xb, hb, acc2, w2, b2, y_o):
    dv = dinv[...]
    tx2 = -2.0 * dv * (p0[...] + p1[...]) - hb[...]
    y_o[...] = jax.nn.relu(xb[...] + acc2[...] + _dot(tx2, w2[...]) + b2[...])


_tc1 = pl.pallas_call(_tc1_body,
                      out_shape=[_f32((_N, _C)), _f32((_N, _C)), _f32((_N, 1))])
_tc3 = pl.pallas_call(_tc3_body, out_shape=[_f32((_N, _C)), _f32((_N, _C))])
_tc5 = pl.pallas_call(_tc5_body, out_shape=[_f32((_N, _C)), _f32((_N, _C))])
_tc9 = pl.pallas_call(_tc9_body, out_shape=_f32((_N, _C)))


def kernel(x, edge_index, gamma1, beta1, W1, b1, gamma2, beta2, W2, b2):
    e = edge_index.shape[1]
    ept = -(-e // _NW)
    nch = -(-ept // _CHUNK)
    nch += nch % 2
    epad = _NW * nch * _CHUNK
    src = edge_index[0]
    dst = edge_index[1]
    pad = epad - e
    if pad:
        ar = jnp.arange(pad, dtype=jnp.int32)
        src = jnp.concatenate([src, ar % _N])
        dst = jnp.concatenate([dst, _N + ar % (_N_ACC - _N)])
    srcp = src.reshape(_NW, nch, _CHUNK)
    dstp = dst.reshape(_NW, nch, _CHUNK)

    deg_k, agg_k = _make_sc_kernels(nch)

    degp = deg_k(dstp)
    xb, z1, dinv = _tc1(x, degp[0, :_N, None], degp[1, :_N, None],
                        gamma1[None], beta1[None])
    ps = agg_k(z1, srcp, dstp)
    acc1, z2 = _tc3(ps[0, :_N], ps[1, :_N], dinv, xb, W1[0], W1[1])
    ps = agg_k(z2, srcp, dstp)
    hb, z3 = _tc5(ps[0, :_N], ps[1, :_N], dinv, xb, acc1, W1[2],
                  b1[None], gamma2[None], beta2[None])
    ps = agg_k(z3, srcp, dstp)
    acc2, z4 = _tc3(ps[0, :_N], ps[1, :_N], dinv, hb, W2[0], W2[1])
    ps = agg_k(z4, srcp, dstp)
    return _tc9(ps[0, :_N], ps[1, :_N], dinv, xb, hb, acc2, W2[2], b2[None])


# speedup vs baseline: 13.2475x; 1.0864x over previous
import functools

import jax
import jax.numpy as jnp
from jax import lax
from jax.experimental import pallas as pl
from jax.experimental.pallas import tpu as pltpu
from jax.experimental.pallas import tpu_sc as plsc

_N = 10000
_C = 128
_EPS = 1e-5

_NC = 2
_NS = 16
_NW = _NC * _NS
_CHUNK = 128
_N_ACC = 10240
_RPT = _N_ACC // _NS
_ZR = 128


def _f32(shape):
    return jax.ShapeDtypeStruct(shape, jnp.float32)


def _deg_body(nch, dst_hbm, out_hbm, idx_d, obuf, zbuf, acc):
    cid = lax.axis_index("c")
    sid = lax.axis_index("s")
    wid = cid * _NS + sid
    one16 = jnp.ones((16,), jnp.float32)
    zero16 = jnp.zeros((16,), jnp.float32)

    def fill(i, carry):
        zbuf[pl.ds(i * 16, 16)] = zero16
        return carry
    lax.fori_loop(0, _RPT // 16, fill, 0)

    def fill1(i, carry):
        obuf[pl.ds(i * 16, 16)] = one16
        return carry
    lax.fori_loop(0, _CHUNK // 16, fill1, 0)

    pltpu.sync_copy(zbuf, acc.at[pl.ds(sid * _RPT, _RPT)])
    plsc.subcore_barrier()
    pltpu.sync_copy(dst_hbm.at[wid], idx_d)

    def body(j, carry):
        pltpu.sync_copy(obuf, acc.at[idx_d.at[j]], add=True)
        return carry
    lax.fori_loop(0, nch, body, 0)
    plsc.subcore_barrier()
    pltpu.sync_copy(acc.at[pl.ds(sid * _RPT, _RPT)],
                    out_hbm.at[cid, pl.ds(sid * _RPT, _RPT)])


def _agg_body(nch, z_hbm, src_hbm, dst_hbm, out_hbm,
              idx_s, idx_d, rows0, rows1, acc, gs0, gs1, ss0, ss1):
    cid = lax.axis_index("c")
    sid = lax.axis_index("s")
    wid = cid * _NS + sid
    zero16 = jnp.zeros((16,), jnp.float32)
    lanes = _C // 16

    def fill(i, carry):
        r = i // lanes
        c0 = (i % lanes) * 16
        rows0[r, pl.ds(c0, 16)] = zero16
        return carry
    lax.fori_loop(0, _CHUNK * lanes, fill, 0)

    for k in range(_RPT // _CHUNK):
        pltpu.sync_copy(rows0, acc.at[pl.ds(sid * _RPT + k * _CHUNK, _CHUNK)])
    plsc.subcore_barrier()

    pltpu.sync_copy(src_hbm.at[wid], idx_s)

    def body(jj, carry):
        j0 = jj * 2
        pltpu.sync_copy(dst_hbm.at[wid, pl.ds(j0, 2)], idx_d)
        g0 = pltpu.async_copy(z_hbm.at[idx_s.at[j0]], rows0, gs0)
        g1 = pltpu.async_copy(z_hbm.at[idx_s.at[j0 + 1]], rows1, gs1)
        g0.wait()
        s0 = pltpu.async_copy(rows0, acc.at[idx_d.at[0]], ss0, add=True)
        g1.wait()
        s1 = pltpu.async_copy(rows1, acc.at[idx_d.at[1]], ss1, add=True)
        s0.wait()
        s1.wait()
        return carry
    lax.fori_loop(0, nch // 2, body, 0)
    plsc.subcore_barrier()
    pltpu.sync_copy(acc.at[pl.ds(sid * _RPT, _RPT)],
                    out_hbm.at[cid, pl.ds(sid * _RPT, _RPT)])


@functools.lru_cache(maxsize=None)
def _make_sc_kernels(nch):
    mesh = plsc.VectorSubcoreMesh(core_axis_name="c", subcore_axis_name="s")
    deg = pl.kernel(
        functools.partial(_deg_body, nch),
        out_type=_f32((_NC, _N_ACC)),
        mesh=mesh,
        scratch_types=[
            pltpu.VMEM((nch, _CHUNK), jnp.int32),
            pltpu.VMEM((_CHUNK,), jnp.float32),
            pltpu.VMEM((_RPT,), jnp.float32),
            pltpu.VMEM_SHARED((_N_ACC,), jnp.float32),
        ],
    )
    agg = pl.kernel(
        functools.partial(_agg_body, nch),
        out_type=_f32((_NC, _N_ACC, _C)),
        mesh=mesh,
        scratch_types=[
            pltpu.VMEM((nch, _CHUNK), jnp.int32),
            pltpu.VMEM((2, _CHUNK), jnp.int32),
            pltpu.VMEM((_CHUNK, _C), jnp.float32),
            pltpu.VMEM((_CHUNK, _C), jnp.float32),
            pltpu.VMEM_SHARED((_N_ACC, _C), jnp.float32),
            pltpu.SemaphoreType.DMA,
            pltpu.SemaphoreType.DMA,
            pltpu.SemaphoreType.DMA,
            pltpu.SemaphoreType.DMA,
        ],
    )
    return deg, agg


def _bn(x, gamma, beta):
    mean = jnp.mean(x, axis=0, keepdims=True)
    var = jnp.mean((x - mean) ** 2, axis=0, keepdims=True)
    return (x - mean) * lax.rsqrt(var + _EPS) * gamma + beta


def _dot(a, b):
    return jnp.dot(a, b, preferred_element_type=jnp.float32)


def _tc1_body(x, d0, d1, g1, b1, xb_o, z1_o, dinv_o):
    deg = d0[...] + d1[...]
    dinv = jnp.where(deg > 0.0, lax.rsqrt(jnp.maximum(deg, 1.0)), 0.0)
    xb = _bn(x[...], g1[...], b1[...])
    xb_o[...] = xb
    z1_o[...] = xb * dinv
    dinv_o[...] = dinv


def _tc3_body(ps, dinv, xin, w0, w1, acc_o, z_o):
    dv = dinv[...]
    tx1 = -dv * (ps[0, :_N] + ps[1, :_N])
    acc_o[...] = _dot(xin[...], w0[...]) + _dot(tx1, w1[...])
    z_o[...] = dv * tx1


def _tc5_body(ps, dinv, xb, acc1, w2, b1, g2, be2, hb_o, z3_o):
    dv = dinv[...]
    tx2 = -2.0 * dv * (ps[0, :_N] + ps[1, :_N]) - xb[...]
    h = jax.nn.relu(acc1[...] + _dot(tx2, w2[...]) + b1[...])
    hb = _bn(h, g2[...], be2[...])
    hb_o[...] = hb
    z3_o[...] = dv * hb


def _tc9_body(ps, dinv, xb, hb, acc2, w2, b2, y_o):
    dv = dinv[...]
    tx2 = -2.0 * dv * (ps[0, :_N] + ps[1, :_N]) - hb[...]
    y_o[...] = jax.nn.relu(xb[...] + acc2[...] + _dot(tx2, w2[...]) + b2[...])


_tc1 = pl.pallas_call(_tc1_body,
                      out_shape=[_f32((_N, _C)), _f32((_N, _C)), _f32((_N, 1))])
_tc3 = pl.pallas_call(_tc3_body, out_shape=[_f32((_N, _C)), _f32((_N, _C))])
_tc5 = pl.pallas_call(_tc5_body, out_shape=[_f32((_N, _C)), _f32((_N, _C))])
_tc9 = pl.pallas_call(_tc9_body, out_shape=_f32((_N, _C)))


def kernel(x, edge_index, gamma1, beta1, W1, b1, gamma2, beta2, W2, b2):
    e = edge_index.shape[1]
    ept = -(-e // _NW)
    nch = -(-ept // _CHUNK)
    nch += nch % 2
    epad = _NW * nch * _CHUNK
    src = edge_index[0]
    dst = edge_index[1]
    pad = epad - e
    if pad:
        ar = jnp.arange(pad, dtype=jnp.int32)
        src = jnp.concatenate([src, ar % _N])
        dst = jnp.concatenate([dst, _N + ar % (_N_ACC - _N)])
    srcp = src.reshape(_NW, nch, _CHUNK)
    dstp = dst.reshape(_NW, nch, _CHUNK)

    deg_k, agg_k = _make_sc_kernels(nch)

    degp = deg_k(dstp)
    xb, z1, dinv = _tc1(x, degp[0, :_N, None], degp[1, :_N, None],
                        gamma1[None], beta1[None])
    ps = agg_k(z1, srcp, dstp)
    acc1, z2 = _tc3(ps, dinv, xb, W1[0], W1[1])
    ps = agg_k(z2, srcp, dstp)
    hb, z3 = _tc5(ps, dinv, xb, acc1, W1[2],
                  b1[None], gamma2[None], beta2[None])
    ps = agg_k(z3, srcp, dstp)
    acc2, z4 = _tc3(ps, dinv, hb, W2[0], W2[1])
    ps = agg_k(z4, srcp, dstp)
    return _tc9(ps, dinv, xb, hb, acc2, W2[2], b2[None])


# speedup vs baseline: 16.1876x; 1.2219x over previous
import functools

import jax
import jax.numpy as jnp
from jax import lax
from jax.experimental import pallas as pl
from jax.experimental.pallas import tpu as pltpu
from jax.experimental.pallas import tpu_sc as plsc

_N = 10000
_C = 128
_EPS = 1e-5

_NC = 2
_NS = 16
_NW = _NC * _NS
_CHUNK = 128
_N_ACC = 10240
_RPT = _N_ACC // _NS
_ZR = 128


def _f32(shape):
    return jax.ShapeDtypeStruct(shape, jnp.float32)


def _deg_body(nch, dst_hbm, out_hbm, idx_d, obuf, zbuf, acc):
    cid = lax.axis_index("c")
    sid = lax.axis_index("s")
    wid = cid * _NS + sid
    one16 = jnp.ones((16,), jnp.float32)
    zero16 = jnp.zeros((16,), jnp.float32)

    def fill(i, carry):
        zbuf[pl.ds(i * 16, 16)] = zero16
        return carry
    lax.fori_loop(0, _RPT // 16, fill, 0)

    def fill1(i, carry):
        obuf[pl.ds(i * 16, 16)] = one16
        return carry
    lax.fori_loop(0, _CHUNK // 16, fill1, 0)

    pltpu.sync_copy(zbuf, acc.at[pl.ds(sid * _RPT, _RPT)])
    plsc.subcore_barrier()
    pltpu.sync_copy(dst_hbm.at[wid], idx_d)

    def body(j, carry):
        pltpu.sync_copy(obuf, acc.at[idx_d.at[j]], add=True)
        return carry
    lax.fori_loop(0, nch, body, 0)
    plsc.subcore_barrier()
    pltpu.sync_copy(acc.at[pl.ds(sid * _RPT, _RPT)],
                    out_hbm.at[cid, pl.ds(sid * _RPT, _RPT)])


def _agg_body(nch, z_hbm, src_hbm, dst_hbm, out_hbm,
              idx_s, idxd0, idxd1, rows0, rows1, acc,
              gs0, gs1, ss0, ss1, ds0, ds1):
    cid = lax.axis_index("c")
    sid = lax.axis_index("s")
    wid = cid * _NS + sid
    zero16 = jnp.zeros((16,), jnp.float32)
    lanes = _C // 16

    def fill(i, carry):
        r = i // lanes
        c0 = (i % lanes) * 16
        rows0[r, pl.ds(c0, 16)] = zero16
        return carry
    lax.fori_loop(0, _CHUNK * lanes, fill, 0)

    for k in range(_RPT // _CHUNK):
        pltpu.sync_copy(rows0, acc.at[pl.ds(sid * _RPT + k * _CHUNK, _CHUNK)])
    plsc.subcore_barrier()

    pltpu.sync_copy(src_hbm.at[wid], idx_s)
    rows = (rows0, rows1)
    idxd = (idxd0, idxd1)
    gsem = (gs0, gs1)
    ssem = (ss0, ss1)
    dsem = (ds0, ds1)

    def gat(j, b):
        pltpu.async_copy(z_hbm.at[idx_s.at[j]], rows[b], gsem[b])

    def sca(j, b):
        pltpu.async_copy(rows[b], acc.at[idxd[b]], ssem[b], add=True)

    def pref(j, b):
        pltpu.async_copy(dst_hbm.at[wid, j], idxd[b], dsem[b])

    def drain_g(b):
        pltpu.make_async_copy(z_hbm.at[pl.ds(0, _CHUNK)], rows[b],
                              gsem[b]).wait()

    def drain_s(b):
        pltpu.make_async_copy(z_hbm.at[pl.ds(0, _CHUNK)], rows[b],
                              ssem[b]).wait()

    def drain_d(b):
        pltpu.make_async_copy(dst_hbm.at[wid, 0], idxd[b], dsem[b]).wait()

    pltpu.sync_copy(dst_hbm.at[wid, 0], idxd0)
    pltpu.sync_copy(dst_hbm.at[wid, 1], idxd1)
    gat(0, 0)
    gat(1, 1)
    drain_g(0)
    sca(0, 0)
    drain_g(1)
    sca(1, 1)
    drain_s(0)
    pref(2, 0)
    gat(2, 0)

    def body(jj, carry):
        for b in range(2):
            j = 2 + jj * 2 + b
            ob = 1 - b
            drain_g(b)
            drain_d(b)
            sca(j, b)
            drain_s(ob)
            pref(j + 1, ob)
            gat(j + 1, ob)
        return carry
    lax.fori_loop(0, (nch - 4) // 2, body, 0)

    drain_g(0)
    drain_d(0)
    sca(nch - 2, 0)
    drain_s(1)
    pref(nch - 1, 1)
    gat(nch - 1, 1)
    drain_g(1)
    drain_d(1)
    sca(nch - 1, 1)
    drain_s(0)
    drain_s(1)

    plsc.subcore_barrier()
    pltpu.sync_copy(acc.at[pl.ds(sid * _RPT, _RPT)],
                    out_hbm.at[cid, pl.ds(sid * _RPT, _RPT)])


@functools.lru_cache(maxsize=None)
def _make_sc_kernels(nch):
    mesh = plsc.VectorSubcoreMesh(core_axis_name="c", subcore_axis_name="s")
    deg = pl.kernel(
        functools.partial(_deg_body, nch),
        out_type=_f32((_NC, _N_ACC)),
        mesh=mesh,
        scratch_types=[
            pltpu.VMEM((nch, _CHUNK), jnp.int32),
            pltpu.VMEM((_CHUNK,), jnp.float32),
            pltpu.VMEM((_RPT,), jnp.float32),
            pltpu.VMEM_SHARED((_N_ACC,), jnp.float32),
        ],
    )
    agg = pl.kernel(
        functools.partial(_agg_body, nch),
        out_type=_f32((_NC, _N_ACC, _C)),
        mesh=mesh,
        scratch_types=[
            pltpu.VMEM((nch, _CHUNK), jnp.int32),
            pltpu.VMEM((_CHUNK,), jnp.int32),
            pltpu.VMEM((_CHUNK,), jnp.int32),
            pltpu.VMEM((_CHUNK, _C), jnp.float32),
            pltpu.VMEM((_CHUNK, _C), jnp.float32),
            pltpu.VMEM_SHARED((_N_ACC, _C), jnp.float32),
            pltpu.SemaphoreType.DMA,
            pltpu.SemaphoreType.DMA,
            pltpu.SemaphoreType.DMA,
            pltpu.SemaphoreType.DMA,
            pltpu.SemaphoreType.DMA,
            pltpu.SemaphoreType.DMA,
        ],
    )
    return deg, agg


def _bn(x, gamma, beta):
    mean = jnp.mean(x, axis=0, keepdims=True)
    var = jnp.mean((x - mean) ** 2, axis=0, keepdims=True)
    return (x - mean) * lax.rsqrt(var + _EPS) * gamma + beta


def _dot(a, b):
    return jnp.dot(a, b, preferred_element_type=jnp.float32)


def _tc1_body(x, d0, d1, g1, b1, xb_o, z1_o, dinv_o):
    deg = d0[...] + d1[...]
    dinv = jnp.where(deg > 0.0, lax.rsqrt(jnp.maximum(deg, 1.0)), 0.0)
    xb = _bn(x[...], g1[...], b1[...])
    xb_o[...] = xb
    z1_o[...] = xb * dinv
    dinv_o[...] = dinv


def _tc3_body(ps, dinv, xin, w0, w1, acc_o, z_o):
    dv = dinv[...]
    tx1 = -dv * (ps[0, :_N] + ps[1, :_N])
    acc_o[...] = _dot(xin[...], w0[...]) + _dot(tx1, w1[...])
    z_o[...] = dv * tx1


def _tc5_body(ps, dinv, xb, acc1, w2, b1, g2, be2, hb_o, z3_o):
    dv = dinv[...]
    tx2 = -2.0 * dv * (ps[0, :_N] + ps[1, :_N]) - xb[...]
    h = jax.nn.relu(acc1[...] + _dot(tx2, w2[...]) + b1[...])
    hb = _bn(h, g2[...], be2[...])
    hb_o[...] = hb
    z3_o[...] = dv * hb


def _tc9_body(ps, dinv, xb, hb, acc2, w2, b2, y_o):
    dv = dinv[...]
    tx2 = -2.0 * dv * (ps[0, :_N] + ps[1, :_N]) - hb[...]
    y_o[...] = jax.nn.relu(xb[...] + acc2[...] + _dot(tx2, w2[...]) + b2[...])


_tc1 = pl.pallas_call(_tc1_body,
                      out_shape=[_f32((_N, _C)), _f32((_N, _C)), _f32((_N, 1))])
_tc3 = pl.pallas_call(_tc3_body, out_shape=[_f32((_N, _C)), _f32((_N, _C))])
_tc5 = pl.pallas_call(_tc5_body, out_shape=[_f32((_N, _C)), _f32((_N, _C))])
_tc9 = pl.pallas_call(_tc9_body, out_shape=_f32((_N, _C)))


def kernel(x, edge_index, gamma1, beta1, W1, b1, gamma2, beta2, W2, b2):
    e = edge_index.shape[1]
    ept = -(-e // _NW)
    nch = -(-ept // _CHUNK)
    nch += nch % 2
    epad = _NW * nch * _CHUNK
    src = edge_index[0]
    dst = edge_index[1]
    pad = epad - e
    if pad:
        ar = jnp.arange(pad, dtype=jnp.int32)
        src = jnp.concatenate([src, ar % _N])
        dst = jnp.concatenate([dst, _N + ar % (_N_ACC - _N)])
    srcp = src.reshape(_NW, nch, _CHUNK)
    dstp = dst.reshape(_NW, nch, _CHUNK)

    deg_k, agg_k = _make_sc_kernels(nch)

    degp = deg_k(dstp)
    xb, z1, dinv = _tc1(x, degp[0, :_N, None], degp[1, :_N, None],
                        gamma1[None], beta1[None])
    ps = agg_k(z1, srcp, dstp)
    acc1, z2 = _tc3(ps, dinv, xb, W1[0], W1[1])
    ps = agg_k(z2, srcp, dstp)
    hb, z3 = _tc5(ps, dinv, xb, acc1, W1[2],
                  b1[None], gamma2[None], beta2[None])
    ps = agg_k(z3, srcp, dstp)
    acc2, z4 = _tc3(ps, dinv, hb, W2[0], W2[1])
    ps = agg_k(z4, srcp, dstp)
    return _tc9(ps, dinv, xb, hb, acc2, W2[2], b2[None])


# speedup vs baseline: 16.4316x; 1.0151x over previous
import functools

import jax
import jax.numpy as jnp
from jax import lax
from jax.experimental import pallas as pl
from jax.experimental.pallas import tpu as pltpu
from jax.experimental.pallas import tpu_sc as plsc

_N = 10000
_C = 128
_EPS = 1e-5

_NC = 2
_NS = 16
_NW = _NC * _NS
_CHUNK = 128
_N_ACC = 10240
_RPT = _N_ACC // _NS
_ZR = 128


def _f32(shape):
    return jax.ShapeDtypeStruct(shape, jnp.float32)


def _deg_body(nch, dst_hbm, out_hbm, idx_d, obuf, zbuf, acc):
    cid = lax.axis_index("c")
    sid = lax.axis_index("s")
    wid = cid * _NS + sid
    one16 = jnp.ones((16,), jnp.float32)
    zero16 = jnp.zeros((16,), jnp.float32)

    def fill(i, carry):
        zbuf[pl.ds(i * 16, 16)] = zero16
        return carry
    lax.fori_loop(0, _RPT // 16, fill, 0)

    def fill1(i, carry):
        obuf[pl.ds(i * 16, 16)] = one16
        return carry
    lax.fori_loop(0, _CHUNK // 16, fill1, 0)

    pltpu.sync_copy(zbuf, acc.at[pl.ds(sid * _RPT, _RPT)])
    plsc.subcore_barrier()
    pltpu.sync_copy(dst_hbm.at[wid], idx_d)

    def body(j, carry):
        pltpu.sync_copy(obuf, acc.at[idx_d.at[j]], add=True)
        return carry
    lax.fori_loop(0, nch, body, 0)
    plsc.subcore_barrier()
    pltpu.sync_copy(acc.at[pl.ds(sid * _RPT, _RPT)],
                    out_hbm.at[cid, pl.ds(sid * _RPT, _RPT)])


def _agg_body(nch, z_hbm, src_hbm, dst_hbm, out_hbm,
              idx_s, idxd0, idxd1, rows0, rows1, acc,
              gs0, gs1, ss0, ss1, ds0, ds1):
    cid = lax.axis_index("c")
    sid = lax.axis_index("s")
    wid = cid * _NS + sid
    zero16 = jnp.zeros((16,), jnp.float32)
    lanes = _C // 16

    def fill(r, carry):
        for c in range(lanes):
            rows0[r, pl.ds(c * 16, 16)] = zero16
        return carry
    lax.fori_loop(0, _CHUNK, fill, 0)

    for k in range(_RPT // _CHUNK):
        pltpu.sync_copy(rows0, acc.at[pl.ds(sid * _RPT + k * _CHUNK, _CHUNK)])
    pltpu.sync_copy(src_hbm.at[wid], idx_s)
    plsc.subcore_barrier()

    rows = (rows0, rows1)
    idxd = (idxd0, idxd1)
    gsem = (gs0, gs1)
    ssem = (ss0, ss1)
    dsem = (ds0, ds1)

    def gat(j, b):
        pltpu.async_copy(z_hbm.at[idx_s.at[j]], rows[b], gsem[b])

    def sca(j, b):
        pltpu.async_copy(rows[b], acc.at[idxd[b]], ssem[b], add=True)

    def pref(j, b):
        pltpu.async_copy(dst_hbm.at[wid, j], idxd[b], dsem[b])

    def drain_g(b):
        pltpu.make_async_copy(z_hbm.at[pl.ds(0, _CHUNK)], rows[b],
                              gsem[b]).wait()

    def drain_s(b):
        pltpu.make_async_copy(z_hbm.at[pl.ds(0, _CHUNK)], rows[b],
                              ssem[b]).wait()

    def drain_d(b):
        pltpu.make_async_copy(dst_hbm.at[wid, 0], idxd[b], dsem[b]).wait()

    pltpu.sync_copy(dst_hbm.at[wid, 0], idxd0)
    pltpu.sync_copy(dst_hbm.at[wid, 1], idxd1)
    gat(0, 0)
    gat(1, 1)
    drain_g(0)
    sca(0, 0)
    drain_g(1)
    sca(1, 1)
    drain_s(0)
    pref(2, 0)
    gat(2, 0)

    def body(jj, carry):
        for b in range(2):
            j = 2 + jj * 2 + b
            ob = 1 - b
            drain_g(b)
            drain_d(b)
            sca(j, b)
            drain_s(ob)
            pref(j + 1, ob)
            gat(j + 1, ob)
        return carry
    lax.fori_loop(0, (nch - 4) // 2, body, 0)

    drain_g(0)
    drain_d(0)
    sca(nch - 2, 0)
    drain_s(1)
    pref(nch - 1, 1)
    gat(nch - 1, 1)
    drain_g(1)
    drain_d(1)
    sca(nch - 1, 1)
    drain_s(0)
    drain_s(1)

    plsc.subcore_barrier()
    pltpu.sync_copy(acc.at[pl.ds(sid * _RPT, _RPT)],
                    out_hbm.at[cid, pl.ds(sid * _RPT, _RPT)])


@functools.lru_cache(maxsize=None)
def _make_sc_kernels(nch):
    mesh = plsc.VectorSubcoreMesh(core_axis_name="c", subcore_axis_name="s")
    deg = pl.kernel(
        functools.partial(_deg_body, nch),
        out_type=_f32((_NC, _N_ACC)),
        mesh=mesh,
        scratch_types=[
            pltpu.VMEM((nch, _CHUNK), jnp.int32),
            pltpu.VMEM((_CHUNK,), jnp.float32),
            pltpu.VMEM((_RPT,), jnp.float32),
            pltpu.VMEM_SHARED((_N_ACC,), jnp.float32),
        ],
    )
    agg = pl.kernel(
        functools.partial(_agg_body, nch),
        out_type=_f32((_NC, _N_ACC, _C)),
        mesh=mesh,
        scratch_types=[
            pltpu.VMEM((nch, _CHUNK), jnp.int32),
            pltpu.VMEM((_CHUNK,), jnp.int32),
            pltpu.VMEM((_CHUNK,), jnp.int32),
            pltpu.VMEM((_CHUNK, _C), jnp.float32),
            pltpu.VMEM((_CHUNK, _C), jnp.float32),
            pltpu.VMEM_SHARED((_N_ACC, _C), jnp.float32),
            pltpu.SemaphoreType.DMA,
            pltpu.SemaphoreType.DMA,
            pltpu.SemaphoreType.DMA,
            pltpu.SemaphoreType.DMA,
            pltpu.SemaphoreType.DMA,
            pltpu.SemaphoreType.DMA,
        ],
    )
    return deg, agg


def _bn(x, gamma, beta):
    mean = jnp.mean(x, axis=0, keepdims=True)
    var = jnp.mean((x - mean) ** 2, axis=0, keepdims=True)
    return (x - mean) * lax.rsqrt(var + _EPS) * gamma + beta


def _dot(a, b):
    return jnp.dot(a, b, preferred_element_type=jnp.float32)


def _tc1_body(x, d0, d1, g1, b1, xb_o, z1_o, dinv_o):
    deg = d0[...] + d1[...]
    dinv = jnp.where(deg > 0.0, lax.rsqrt(jnp.maximum(deg, 1.0)), 0.0)
    xb = _bn(x[...], g1[...], b1[...])
    xb_o[...] = xb
    z1_o[...] = xb * dinv
    dinv_o[...] = dinv


def _tc3_body(ps, dinv, xin, w0, w1, acc_o, z_o):
    dv = dinv[...]
    tx1 = -dv * (ps[0] + ps[1])
    acc_o[...] = _dot(xin[...], w0[...]) + _dot(tx1, w1[...])
    z_o[...] = dv * tx1


def _tc5_body(ps, dinv, xb, acc1, w2, b1, g2, be2, hb_o, z3_o):
    dv = dinv[...]
    tx2 = -2.0 * dv * (ps[0, :_N] + ps[1, :_N]) - xb[...]
    h = jax.nn.relu(acc1[...] + _dot(tx2, w2[...]) + b1[...])
    hb = _bn(h, g2[...], be2[...])
    hb_o[...] = hb
    z3_o[...] = dv * hb


def _tc9_body(ps, dinv, xb, hb, acc2, w2, b2, y_o):
    dv = dinv[...]
    tx2 = -2.0 * dv * (ps[0] + ps[1]) - hb[...]
    y_o[...] = jax.nn.relu(xb[...] + acc2[...] + _dot(tx2, w2[...]) + b2[...])


_G = 10
_BR = _N // _G

_ps_spec = pl.BlockSpec((_NC, _BR, _C), lambda i: (0, i, 0))
_row_spec = pl.BlockSpec((_BR, _C), lambda i: (i, 0))
_col_spec = pl.BlockSpec((_BR, 1), lambda i: (i, 0))
_w_spec = pl.BlockSpec((_C, _C), lambda i: (0, 0))
_b_spec = pl.BlockSpec((1, _C), lambda i: (0, 0))

_tc1 = pl.pallas_call(_tc1_body,
                      out_shape=[_f32((_N, _C)), _f32((_N, _C)), _f32((_N, 1))])
_tc3 = pl.pallas_call(
    _tc3_body, grid=(_G,),
    in_specs=[_ps_spec, _col_spec, _row_spec, _w_spec, _w_spec],
    out_specs=[_row_spec, _row_spec],
    out_shape=[_f32((_N, _C)), _f32((_N, _C))])
_tc5 = pl.pallas_call(_tc5_body, out_shape=[_f32((_N, _C)), _f32((_N, _C))])
_tc9 = pl.pallas_call(
    _tc9_body, grid=(_G,),
    in_specs=[_ps_spec, _col_spec, _row_spec, _row_spec, _row_spec,
              _w_spec, _b_spec],
    out_specs=_row_spec,
    out_shape=_f32((_N, _C)))


def kernel(x, edge_index, gamma1, beta1, W1, b1, gamma2, beta2, W2, b2):
    e = edge_index.shape[1]
    ept = -(-e // _NW)
    nch = -(-ept // _CHUNK)
    nch += nch % 2
    epad = _NW * nch * _CHUNK
    src = edge_index[0]
    dst = edge_index[1]
    pad = epad - e
    if pad:
        ar = jnp.arange(pad, dtype=jnp.int32)
        src = jnp.concatenate([src, ar % _N])
        dst = jnp.concatenate([dst, _N + ar % (_N_ACC - _N)])
    srcp = src.reshape(_NW, nch, _CHUNK)
    dstp = dst.reshape(_NW, nch, _CHUNK)

    deg_k, agg_k = _make_sc_kernels(nch)

    degp = deg_k(dstp)
    xb, z1, dinv = _tc1(x, degp[0, :_N, None], degp[1, :_N, None],
                        gamma1[None], beta1[None])
    ps = agg_k(z1, srcp, dstp)
    acc1, z2 = _tc3(ps, dinv, xb, W1[0], W1[1])
    ps = agg_k(z2, srcp, dstp)
    hb, z3 = _tc5(ps, dinv, xb, acc1, W1[2],
                  b1[None], gamma2[None], beta2[None])
    ps = agg_k(z3, srcp, dstp)
    acc2, z4 = _tc3(ps, dinv, hb, W2[0], W2[1])
    ps = agg_k(z4, srcp, dstp)
    return _tc9(ps, dinv, xb, hb, acc2, W2[2], b2[None])


# speedup vs baseline: 16.5723x; 1.0086x over previous
import functools

import jax
import jax.numpy as jnp
from jax import lax
from jax.experimental import pallas as pl
from jax.experimental.pallas import tpu as pltpu
from jax.experimental.pallas import tpu_sc as plsc

_N = 10000
_C = 128
_EPS = 1e-5

_NC = 2
_NS = 16
_NW = _NC * _NS
_CHUNK = 128
_N_ACC = 10240
_RPT = _N_ACC // _NS
_ZR = 128


def _f32(shape):
    return jax.ShapeDtypeStruct(shape, jnp.float32)


def _deg_body(nch, dst_hbm, out_hbm, idx_d, obuf, zbuf, acc):
    cid = lax.axis_index("c")
    sid = lax.axis_index("s")
    wid = cid * _NS + sid
    one16 = jnp.ones((16,), jnp.float32)
    zero16 = jnp.zeros((16,), jnp.float32)

    def fill(i, carry):
        zbuf[pl.ds(i * 16, 16)] = zero16
        return carry
    lax.fori_loop(0, _RPT // 16, fill, 0)

    def fill1(i, carry):
        obuf[pl.ds(i * 16, 16)] = one16
        return carry
    lax.fori_loop(0, _CHUNK // 16, fill1, 0)

    pltpu.sync_copy(zbuf, acc.at[pl.ds(sid * _RPT, _RPT)])
    plsc.subcore_barrier()
    pltpu.sync_copy(dst_hbm.at[wid], idx_d)

    def body(j, carry):
        pltpu.sync_copy(obuf, acc.at[idx_d.at[j]], add=True)
        return carry
    lax.fori_loop(0, nch, body, 0)
    plsc.subcore_barrier()
    pltpu.sync_copy(acc.at[pl.ds(sid * _RPT, _RPT)],
                    out_hbm.at[cid, pl.ds(sid * _RPT, _RPT)])


def _agg_body(nch, z_hbm, src_hbm, dst_hbm, out_hbm,
              idx_s, idxd0, idxd1, rows0, rows1, acc,
              gs0, gs1, ss0, ss1, ds0, ds1):
    cid = lax.axis_index("c")
    sid = lax.axis_index("s")
    wid = cid * _NS + sid
    zero16 = jnp.zeros((16,), jnp.float32)
    lanes = _C // 16

    def fill(r, carry):
        for c in range(lanes):
            rows0[r, pl.ds(c * 16, 16)] = zero16
        return carry
    lax.fori_loop(0, _CHUNK, fill, 0)

    for k in range(_RPT // _CHUNK):
        pltpu.sync_copy(rows0, acc.at[pl.ds(sid * _RPT + k * _CHUNK, _CHUNK)])
    pltpu.sync_copy(src_hbm.at[wid], idx_s)
    plsc.subcore_barrier()

    rows = (rows0, rows1)
    idxd = (idxd0, idxd1)
    gsem = (gs0, gs1)
    ssem = (ss0, ss1)
    dsem = (ds0, ds1)

    def gat(j, b):
        pltpu.async_copy(z_hbm.at[idx_s.at[j]], rows[b], gsem[b])

    def sca(j, b):
        pltpu.async_copy(rows[b], acc.at[idxd[b]], ssem[b], add=True)

    def pref(j, b):
        pltpu.async_copy(dst_hbm.at[wid, j], idxd[b], dsem[b])

    def drain_g(b):
        pltpu.make_async_copy(z_hbm.at[pl.ds(0, _CHUNK)], rows[b],
                              gsem[b]).wait()

    def drain_s(b):
        pltpu.make_async_copy(z_hbm.at[pl.ds(0, _CHUNK)], rows[b],
                              ssem[b]).wait()

    def drain_d(b):
        pltpu.make_async_copy(dst_hbm.at[wid, 0], idxd[b], dsem[b]).wait()

    pltpu.sync_copy(dst_hbm.at[wid, 0], idxd0)
    pltpu.sync_copy(dst_hbm.at[wid, 1], idxd1)
    gat(0, 0)
    gat(1, 1)
    drain_g(0)
    sca(0, 0)
    drain_g(1)
    sca(1, 1)
    drain_s(0)
    pref(2, 0)
    gat(2, 0)

    def body(jj, carry):
        for b in range(2):
            j = 2 + jj * 2 + b
            ob = 1 - b
            drain_g(b)
            drain_d(b)
            sca(j, b)
            drain_s(ob)
            pref(j + 1, ob)
            gat(j + 1, ob)
        return carry
    lax.fori_loop(0, (nch - 4) // 2, body, 0)

    drain_g(0)
    drain_d(0)
    sca(nch - 2, 0)
    drain_s(1)
    pref(nch - 1, 1)
    gat(nch - 1, 1)
    drain_g(1)
    drain_d(1)
    sca(nch - 1, 1)
    drain_s(0)
    drain_s(1)

    plsc.subcore_barrier()
    pltpu.sync_copy(acc.at[pl.ds(sid * _RPT, _RPT)],
                    out_hbm.at[cid, pl.ds(sid * _RPT, _RPT)])


@functools.lru_cache(maxsize=None)
def _make_sc_kernels(nch):
    mesh = plsc.VectorSubcoreMesh(core_axis_name="c", subcore_axis_name="s")
    deg = pl.kernel(
        functools.partial(_deg_body, nch),
        out_type=_f32((_NC, _N_ACC)),
        mesh=mesh,
        scratch_types=[
            pltpu.VMEM((nch, _CHUNK), jnp.int32),
            pltpu.VMEM((_CHUNK,), jnp.float32),
            pltpu.VMEM((_RPT,), jnp.float32),
            pltpu.VMEM_SHARED((_N_ACC,), jnp.float32),
        ],
    )
    agg = pl.kernel(
        functools.partial(_agg_body, nch),
        out_type=_f32((_NC, _N_ACC, _C)),
        mesh=mesh,
        scratch_types=[
            pltpu.VMEM((nch, _CHUNK), jnp.int32),
            pltpu.VMEM((_CHUNK,), jnp.int32),
            pltpu.VMEM((_CHUNK,), jnp.int32),
            pltpu.VMEM((_CHUNK, _C), jnp.float32),
            pltpu.VMEM((_CHUNK, _C), jnp.float32),
            pltpu.VMEM_SHARED((_N_ACC, _C), jnp.float32),
            pltpu.SemaphoreType.DMA,
            pltpu.SemaphoreType.DMA,
            pltpu.SemaphoreType.DMA,
            pltpu.SemaphoreType.DMA,
            pltpu.SemaphoreType.DMA,
            pltpu.SemaphoreType.DMA,
        ],
    )
    return deg, agg


def _bn(x, gamma, beta):
    mean = jnp.mean(x, axis=0, keepdims=True)
    var = jnp.mean((x - mean) ** 2, axis=0, keepdims=True)
    return (x - mean) * lax.rsqrt(var + _EPS) * gamma + beta


def _dot(a, b):
    return jnp.dot(a, b, preferred_element_type=jnp.float32)


def _tcbn_body(x, g, b, xb_o):
    xb_o[...] = _bn(x[...], g[...], b[...])


def _tcz_body(d0, d1, xb, z_o, dinv_o):
    deg = d0[...] + d1[...]
    dinv = jnp.where(deg > 0.0, lax.rsqrt(jnp.maximum(deg, 1.0)), 0.0)
    z_o[...] = xb[...] * dinv
    dinv_o[...] = dinv


def _tca_body(ps, dinv, q_o, z_o):
    dv = dinv[...]
    q = dv * (ps[0] + ps[1])
    q_o[...] = q
    z_o[...] = -dv * q


def _tcb_body(xin, q, w0, w1, w2, m_o):
    m_o[...] = _dot(xin[...], w0[...] - w2[...]) - _dot(q[...], w1[...])


def _tcc_body(ps, dinv, m1, w2, b1, g2, be2, hb_o, z_o):
    dv = dinv[...]
    txw = -2.0 * dv * (ps[0, :_N] + ps[1, :_N])
    h = jax.nn.relu(m1[...] + _dot(txw, w2[...]) + b1[...])
    hb = _bn(h, g2[...], be2[...])
    hb_o[...] = hb
    z_o[...] = dv * hb


def _tcf_body(ps, dinv, xb, m2, w2, b2, y_o):
    dv = dinv[...]
    txw = -2.0 * dv * (ps[0] + ps[1])
    y_o[...] = jax.nn.relu(xb[...] + m2[...] + _dot(txw, w2[...]) + b2[...])


_G = 10
_BR = _N // _G

_ps_spec = pl.BlockSpec((_NC, _BR, _C), lambda i: (0, i, 0))
_row_spec = pl.BlockSpec((_BR, _C), lambda i: (i, 0))
_col_spec = pl.BlockSpec((_BR, 1), lambda i: (i, 0))
_w_spec = pl.BlockSpec((_C, _C), lambda i: (0, 0))
_b_spec = pl.BlockSpec((1, _C), lambda i: (0, 0))

_tcbn = pl.pallas_call(_tcbn_body, out_shape=_f32((_N, _C)))
_tcz = pl.pallas_call(
    _tcz_body, grid=(_G,),
    in_specs=[_col_spec, _col_spec, _row_spec],
    out_specs=[_row_spec, _col_spec],
    out_shape=[_f32((_N, _C)), _f32((_N, 1))])
_tca = pl.pallas_call(
    _tca_body, grid=(_G,),
    in_specs=[_ps_spec, _col_spec],
    out_specs=[_row_spec, _row_spec],
    out_shape=[_f32((_N, _C)), _f32((_N, _C))])
_tcb = pl.pallas_call(
    _tcb_body, grid=(_G,),
    in_specs=[_row_spec, _row_spec, _w_spec, _w_spec, _w_spec],
    out_specs=_row_spec,
    out_shape=_f32((_N, _C)))
_tcc = pl.pallas_call(_tcc_body, out_shape=[_f32((_N, _C)), _f32((_N, _C))])
_tcf = pl.pallas_call(
    _tcf_body, grid=(_G,),
    in_specs=[_ps_spec, _col_spec, _row_spec, _row_spec, _w_spec, _b_spec],
    out_specs=_row_spec,
    out_shape=_f32((_N, _C)))


def kernel(x, edge_index, gamma1, beta1, W1, b1, gamma2, beta2, W2, b2):
    e = edge_index.shape[1]
    ept = -(-e // _NW)
    nch = -(-ept // _CHUNK)
    nch += nch % 2
    epad = _NW * nch * _CHUNK
    src = edge_index[0]
    dst = edge_index[1]
    pad = epad - e
    if pad:
        ar = jnp.arange(pad, dtype=jnp.int32)
        src = jnp.concatenate([src, ar % _N])
        dst = jnp.concatenate([dst, _N + ar % (_N_ACC - _N)])
    srcp = src.reshape(_NW, nch, _CHUNK)
    dstp = dst.reshape(_NW, nch, _CHUNK)

    deg_k, agg_k = _make_sc_kernels(nch)

    degp = deg_k(dstp)
    xb = _tcbn(x, gamma1[None], beta1[None])
    z1, dinv = _tcz(degp[0, :_N, None], degp[1, :_N, None], xb)
    ps = agg_k(z1, srcp, dstp)
    q1d, z2 = _tca(ps, dinv)
    m1 = _tcb(xb, q1d, W1[0], W1[1], W1[2])
    ps = agg_k(z2, srcp, dstp)
    hb, z3 = _tcc(ps, dinv, m1, W1[2], b1[None], gamma2[None], beta2[None])
    ps = agg_k(z3, srcp, dstp)
    q3d, z4 = _tca(ps, dinv)
    m2 = _tcb(hb, q3d, W2[0], W2[1], W2[2])
    ps = agg_k(z4, srcp, dstp)
    return _tcf(ps, dinv, xb, m2, W2[2], b2[None])


# speedup vs baseline: 16.6789x; 1.0064x over previous
import functools

import jax
import jax.numpy as jnp
from jax import lax
from jax.experimental import pallas as pl
from jax.experimental.pallas import tpu as pltpu
from jax.experimental.pallas import tpu_sc as plsc

_N = 10000
_C = 128
_EPS = 1e-5

_NC = 2
_NS = 16
_NW = _NC * _NS
_CHUNK = 128
_N_ACC = 10240
_RPT = _N_ACC // _NS
_ZR = 128


def _f32(shape):
    return jax.ShapeDtypeStruct(shape, jnp.float32)


def _deg_body(nch, dst_hbm, out_hbm, idx_d, obuf, zbuf, acc):
    cid = lax.axis_index("c")
    sid = lax.axis_index("s")
    wid = cid * _NS + sid
    one16 = jnp.ones((16,), jnp.float32)
    zero16 = jnp.zeros((16,), jnp.float32)

    def fill(i, carry):
        zbuf[pl.ds(i * 16, 16)] = zero16
        return carry
    lax.fori_loop(0, _RPT // 16, fill, 0)

    def fill1(i, carry):
        obuf[pl.ds(i * 16, 16)] = one16
        return carry
    lax.fori_loop(0, _CHUNK // 16, fill1, 0)

    pltpu.sync_copy(zbuf, acc.at[pl.ds(sid * _RPT, _RPT)])
    plsc.subcore_barrier()
    pltpu.sync_copy(dst_hbm.at[wid], idx_d)

    def body(j, carry):
        pltpu.sync_copy(obuf, acc.at[idx_d.at[j]], add=True)
        return carry
    lax.fori_loop(0, nch, body, 0)
    plsc.subcore_barrier()
    pltpu.sync_copy(acc.at[pl.ds(sid * _RPT, _RPT)],
                    out_hbm.at[cid, pl.ds(sid * _RPT, _RPT)])


def _agg_body(nch, z_hbm, src_hbm, dst_hbm, out_hbm,
              idx_s, idxd0, idxd1, rows0, rows1, acc,
              gs0, gs1, ss0, ss1, ds0, ds1):
    cid = lax.axis_index("c")
    sid = lax.axis_index("s")
    wid = cid * _NS + sid
    zero16 = jnp.zeros((16,), jnp.float32)
    lanes = _C // 16

    def fill(r, carry):
        for c in range(lanes):
            rows0[r, pl.ds(c * 16, 16)] = zero16
        return carry
    lax.fori_loop(0, _CHUNK, fill, 0)

    for k in range(_RPT // _CHUNK):
        pltpu.async_copy(rows0,
                         acc.at[pl.ds(sid * _RPT + k * _CHUNK, _CHUNK)], ss0)
    pltpu.async_copy(src_hbm.at[wid], idx_s, gs0)
    for k in range(_RPT // _CHUNK):
        pltpu.make_async_copy(
            rows0, acc.at[pl.ds(sid * _RPT + k * _CHUNK, _CHUNK)], ss0).wait()
    pltpu.make_async_copy(src_hbm.at[wid], idx_s, gs0).wait()
    plsc.subcore_barrier()

    rows = (rows0, rows1)
    idxd = (idxd0, idxd1)
    gsem = (gs0, gs1)
    ssem = (ss0, ss1)
    dsem = (ds0, ds1)

    def gat(j, b):
        pltpu.async_copy(z_hbm.at[idx_s.at[j]], rows[b], gsem[b])

    def sca(j, b):
        pltpu.async_copy(rows[b], acc.at[idxd[b]], ssem[b], add=True)

    def pref(j, b):
        pltpu.async_copy(dst_hbm.at[wid, j], idxd[b], dsem[b])

    def drain_g(b):
        pltpu.make_async_copy(z_hbm.at[pl.ds(0, _CHUNK)], rows[b],
                              gsem[b]).wait()

    def drain_s(b):
        pltpu.make_async_copy(z_hbm.at[pl.ds(0, _CHUNK)], rows[b],
                              ssem[b]).wait()

    def drain_d(b):
        pltpu.make_async_copy(dst_hbm.at[wid, 0], idxd[b], dsem[b]).wait()

    pltpu.sync_copy(dst_hbm.at[wid, 0], idxd0)
    pltpu.sync_copy(dst_hbm.at[wid, 1], idxd1)
    gat(0, 0)
    gat(1, 1)
    drain_g(0)
    sca(0, 0)
    drain_g(1)
    sca(1, 1)
    drain_s(0)
    pref(2, 0)
    gat(2, 0)

    def body(jj, carry):
        for b in range(2):
            j = 2 + jj * 2 + b
            ob = 1 - b
            drain_g(b)
            drain_d(b)
            sca(j, b)
            drain_s(ob)
            pref(j + 1, ob)
            gat(j + 1, ob)
        return carry
    lax.fori_loop(0, (nch - 4) // 2, body, 0)

    drain_g(0)
    drain_d(0)
    sca(nch - 2, 0)
    drain_s(1)
    pref(nch - 1, 1)
    gat(nch - 1, 1)
    drain_g(1)
    drain_d(1)
    sca(nch - 1, 1)
    drain_s(0)
    drain_s(1)

    plsc.subcore_barrier()
    pltpu.sync_copy(acc.at[pl.ds(sid * _RPT, _RPT)],
                    out_hbm.at[cid, pl.ds(sid * _RPT, _RPT)])


@functools.lru_cache(maxsize=None)
def _make_sc_kernels(nch):
    mesh = plsc.VectorSubcoreMesh(core_axis_name="c", subcore_axis_name="s")
    deg = pl.kernel(
        functools.partial(_deg_body, nch),
        out_type=_f32((_NC, _N_ACC)),
        mesh=mesh,
        scratch_types=[
            pltpu.VMEM((nch, _CHUNK), jnp.int32),
            pltpu.VMEM((_CHUNK,), jnp.float32),
            pltpu.VMEM((_RPT,), jnp.float32),
            pltpu.VMEM_SHARED((_N_ACC,), jnp.float32),
        ],
    )
    agg = pl.kernel(
        functools.partial(_agg_body, nch),
        out_type=_f32((_NC, _N_ACC, _C)),
        mesh=mesh,
        scratch_types=[
            pltpu.VMEM((nch, _CHUNK), jnp.int32),
            pltpu.VMEM((_CHUNK,), jnp.int32),
            pltpu.VMEM((_CHUNK,), jnp.int32),
            pltpu.VMEM((_CHUNK, _C), jnp.float32),
            pltpu.VMEM((_CHUNK, _C), jnp.float32),
            pltpu.VMEM_SHARED((_N_ACC, _C), jnp.float32),
            pltpu.SemaphoreType.DMA,
            pltpu.SemaphoreType.DMA,
            pltpu.SemaphoreType.DMA,
            pltpu.SemaphoreType.DMA,
            pltpu.SemaphoreType.DMA,
            pltpu.SemaphoreType.DMA,
        ],
    )
    return deg, agg


def _bn(x, gamma, beta):
    mean = jnp.mean(x, axis=0, keepdims=True)
    var = jnp.mean((x - mean) ** 2, axis=0, keepdims=True)
    return (x - mean) * lax.rsqrt(var + _EPS) * gamma + beta


def _dot(a, b):
    return jnp.dot(a, b, preferred_element_type=jnp.float32)


def _tcbn_body(x, g, b, xb_o):
    xb_o[...] = _bn(x[...], g[...], b[...])


def _tcz_body(d0, d1, xb, z_o, dinv_o):
    deg = d0[...] + d1[...]
    dinv = jnp.where(deg > 0.0, lax.rsqrt(jnp.maximum(deg, 1.0)), 0.0)
    z_o[...] = xb[...] * dinv
    dinv_o[...] = dinv


def _tca_body(ps, dinv, q_o, z_o):
    dv = dinv[...]
    q = dv * (ps[0] + ps[1])
    q_o[...] = q
    z_o[...] = -dv * q


def _tcb_body(xin, q, w0, w1, w2, m_o):
    m_o[...] = _dot(xin[...], w0[...] - w2[...]) - _dot(q[...], w1[...])


def _tcc_body(ps, dinv, m1, w2, b1, g2, be2, hb_o, z_o):
    dv = dinv[...]
    txw = -2.0 * dv * (ps[0, :_N] + ps[1, :_N])
    h = jax.nn.relu(m1[...] + _dot(txw, w2[...]) + b1[...])
    hb = _bn(h, g2[...], be2[...])
    hb_o[...] = hb
    z_o[...] = dv * hb


def _tcf_body(ps, dinv, xb, m2, w2, b2, y_o):
    dv = dinv[...]
    txw = -2.0 * dv * (ps[0] + ps[1])
    y_o[...] = jax.nn.relu(xb[...] + m2[...] + _dot(txw, w2[...]) + b2[...])


_G = 10
_BR = _N // _G

_ps_spec = pl.BlockSpec((_NC, _BR, _C), lambda i: (0, i, 0))
_row_spec = pl.BlockSpec((_BR, _C), lambda i: (i, 0))
_col_spec = pl.BlockSpec((_BR, 1), lambda i: (i, 0))
_w_spec = pl.BlockSpec((_C, _C), lambda i: (0, 0))
_b_spec = pl.BlockSpec((1, _C), lambda i: (0, 0))

_tcbn = pl.pallas_call(_tcbn_body, out_shape=_f32((_N, _C)))
_tcz = pl.pallas_call(
    _tcz_body, grid=(_G,),
    in_specs=[_col_spec, _col_spec, _row_spec],
    out_specs=[_row_spec, _col_spec],
    out_shape=[_f32((_N, _C)), _f32((_N, 1))])
_tca = pl.pallas_call(
    _tca_body, grid=(_G,),
    in_specs=[_ps_spec, _col_spec],
    out_specs=[_row_spec, _row_spec],
    out_shape=[_f32((_N, _C)), _f32((_N, _C))])
_tcb = pl.pallas_call(
    _tcb_body, grid=(_G,),
    in_specs=[_row_spec, _row_spec, _w_spec, _w_spec, _w_spec],
    out_specs=_row_spec,
    out_shape=_f32((_N, _C)))
_tcc = pl.pallas_call(_tcc_body, out_shape=[_f32((_N, _C)), _f32((_N, _C))])
_tcf = pl.pallas_call(
    _tcf_body, grid=(_G,),
    in_specs=[_ps_spec, _col_spec, _row_spec, _row_spec, _w_spec, _b_spec],
    out_specs=_row_spec,
    out_shape=_f32((_N, _C)))


def kernel(x, edge_index, gamma1, beta1, W1, b1, gamma2, beta2, W2, b2):
    e = edge_index.shape[1]
    ept = -(-e // _NW)
    nch = -(-ept // _CHUNK)
    nch += nch % 2
    epad = _NW * nch * _CHUNK
    src = edge_index[0]
    dst = edge_index[1]
    pad = epad - e
    if pad:
        ar = jnp.arange(pad, dtype=jnp.int32)
        src = jnp.concatenate([src, ar % _N])
        dst = jnp.concatenate([dst, _N + ar % (_N_ACC - _N)])
    srcp = src.reshape(_NW, nch, _CHUNK)
    dstp = dst.reshape(_NW, nch, _CHUNK)

    deg_k, agg_k = _make_sc_kernels(nch)

    degp = deg_k(dstp)
    xb = _tcbn(x, gamma1[None], beta1[None])
    z1, dinv = _tcz(degp[0, :_N, None], degp[1, :_N, None], xb)
    ps = agg_k(z1, srcp, dstp)
    q1d, z2 = _tca(ps, dinv)
    m1 = _tcb(xb, q1d, W1[0], W1[1], W1[2])
    ps = agg_k(z2, srcp, dstp)
    hb, z3 = _tcc(ps, dinv, m1, W1[2], b1[None], gamma2[None], beta2[None])
    ps = agg_k(z3, srcp, dstp)
    q3d, z4 = _tca(ps, dinv)
    m2 = _tcb(hb, q3d, W2[0], W2[1], W2[2])
    ps = agg_k(z4, srcp, dstp)
    return _tcf(ps, dinv, xb, m2, W2[2], b2[None])


# speedup vs baseline: 16.8484x; 1.0102x over previous
import functools

import jax
import jax.numpy as jnp
from jax import lax
from jax.experimental import pallas as pl
from jax.experimental.pallas import tpu as pltpu
from jax.experimental.pallas import tpu_sc as plsc

_N = 10000
_C = 128
_EPS = 1e-5

_NC = 2
_NS = 16
_NW = _NC * _NS
_CHUNK = 128
_N_ACC = 10112
_RPT = _N_ACC // _NS
_N_DEG = 10240
_RPT_DEG = _N_DEG // _NS


def _f32(shape):
    return jax.ShapeDtypeStruct(shape, jnp.float32)


def _deg_body(nch, dst_hbm, out_hbm, idx_d, obuf, zbuf, acc):
    cid = lax.axis_index("c")
    sid = lax.axis_index("s")
    wid = cid * _NS + sid
    one16 = jnp.ones((16,), jnp.float32)
    zero16 = jnp.zeros((16,), jnp.float32)

    def fill(i, carry):
        zbuf[pl.ds(i * 16, 16)] = zero16
        return carry
    lax.fori_loop(0, _RPT_DEG // 16, fill, 0)

    def fill1(i, carry):
        obuf[pl.ds(i * 16, 16)] = one16
        return carry
    lax.fori_loop(0, _CHUNK // 16, fill1, 0)

    pltpu.sync_copy(zbuf, acc.at[pl.ds(sid * _RPT_DEG, _RPT_DEG)])
    plsc.subcore_barrier()
    pltpu.sync_copy(dst_hbm.at[wid], idx_d)

    def body(j, carry):
        pltpu.sync_copy(obuf, acc.at[idx_d.at[j]], add=True)
        return carry
    lax.fori_loop(0, nch, body, 0)
    plsc.subcore_barrier()
    pltpu.sync_copy(acc.at[pl.ds(sid * _RPT_DEG, _RPT_DEG)],
                    out_hbm.at[cid, pl.ds(sid * _RPT_DEG, _RPT_DEG)])


def _agg_body(nch, z_hbm, src_hbm, dst_hbm, out_hbm,
              idxs0, idxs1, idxs2, idxd0, idxd1, idxd2,
              rows0, rows1, rows2, acc,
              gs0, gs1, gs2, ss0, ss1, ss2,
              is0, is1, is2, id0, id1, id2):
    cid = lax.axis_index("c")
    sid = lax.axis_index("s")
    wid = cid * _NS + sid
    zero16 = jnp.zeros((16,), jnp.float32)
    lanes = _C // 16

    def fill(r, carry):
        for c in range(lanes):
            rows0[r, pl.ds(c * 16, 16)] = zero16
        return carry
    lax.fori_loop(0, _CHUNK, fill, 0)

    nz = _RPT // _CHUNK
    rz = _RPT - nz * _CHUNK
    for k in range(nz):
        pltpu.async_copy(rows0,
                         acc.at[pl.ds(sid * _RPT + k * _CHUNK, _CHUNK)], ss0)
    if rz:
        pltpu.async_copy(rows0.at[pl.ds(0, rz)],
                         acc.at[pl.ds(sid * _RPT + nz * _CHUNK, rz)], ss1)
    for k in range(nz):
        pltpu.make_async_copy(
            rows0, acc.at[pl.ds(sid * _RPT + k * _CHUNK, _CHUNK)], ss0).wait()
    if rz:
        pltpu.make_async_copy(
            rows0.at[pl.ds(0, rz)],
            acc.at[pl.ds(sid * _RPT + nz * _CHUNK, rz)], ss1).wait()
    plsc.subcore_barrier()

    rows = (rows0, rows1, rows2)
    idxs = (idxs0, idxs1, idxs2)
    idxd = (idxd0, idxd1, idxd2)
    gsem = (gs0, gs1, gs2)
    ssem = (ss0, ss1, ss2)
    isem = (is0, is1, is2)
    dsem = (id0, id1, id2)

    def gat(j, b):
        pltpu.async_copy(z_hbm.at[idxs[b]], rows[b], gsem[b])

    def sca(j, b):
        pltpu.async_copy(rows[b], acc.at[idxd[b]], ssem[b], add=True)

    ibase = wid * (nch * _CHUNK)

    def pref_s(j, b):
        pltpu.async_copy(src_hbm.at[pl.ds(ibase + j * _CHUNK, _CHUNK)],
                         idxs[b], isem[b])

    def pref_d(j, b):
        pltpu.async_copy(dst_hbm.at[pl.ds(ibase + j * _CHUNK, _CHUNK)],
                         idxd[b], dsem[b])

    def drain_g(b):
        pltpu.make_async_copy(z_hbm.at[pl.ds(0, _CHUNK)], rows[b],
                              gsem[b]).wait()

    def drain_s(b):
        pltpu.make_async_copy(z_hbm.at[pl.ds(0, _CHUNK)], rows[b],
                              ssem[b]).wait()

    def drain_is(b):
        pltpu.make_async_copy(src_hbm.at[pl.ds(0, _CHUNK)], idxs[b],
                              isem[b]).wait()

    def drain_id(b):
        pltpu.make_async_copy(dst_hbm.at[pl.ds(0, _CHUNK)], idxd[b],
                              dsem[b]).wait()

    pltpu.sync_copy(src_hbm.at[pl.ds(ibase, _CHUNK)], idxs0)
    pltpu.sync_copy(src_hbm.at[pl.ds(ibase + _CHUNK, _CHUNK)], idxs1)
    pltpu.sync_copy(dst_hbm.at[pl.ds(ibase, _CHUNK)], idxd0)
    pltpu.sync_copy(dst_hbm.at[pl.ds(ibase + _CHUNK, _CHUNK)], idxd1)
    gat(0, 0)
    gat(1, 1)
    pref_s(2, 2)
    pref_d(2, 2)
    drain_g(0)
    sca(0, 0)
    drain_g(1)
    sca(1, 1)
    drain_is(2)
    gat(2, 2)
    pref_s(3, 0)

    def body(jj, carry):
        for u in range(3):
            j = 2 + jj * 3 + u
            b = (2 + u) % 3
            bn1 = (b + 1) % 3
            bn2 = (b + 2) % 3
            drain_g(b)
            drain_id(b)
            sca(j, b)
            drain_s(bn1)
            pref_d(j + 1, bn1)
            drain_is(bn1)
            gat(j + 1, bn1)
            pref_s(j + 2, bn2)
        return carry
    lax.fori_loop(0, (nch - 4) // 3, body, 0)

    b = (nch - 2) % 3
    bn1 = (b + 1) % 3
    drain_g(b)
    drain_id(b)
    sca(nch - 2, b)
    drain_s(bn1)
    pref_d(nch - 1, bn1)
    drain_is(bn1)
    gat(nch - 1, bn1)
    b = (nch - 1) % 3
    drain_g(b)
    drain_id(b)
    sca(nch - 1, b)
    drain_s((nch - 3) % 3)
    drain_s((nch - 2) % 3)
    drain_s((nch - 1) % 3)

    plsc.subcore_barrier()
    pltpu.sync_copy(acc.at[pl.ds(sid * _RPT, _RPT)],
                    out_hbm.at[cid, pl.ds(sid * _RPT, _RPT)])


@functools.lru_cache(maxsize=None)
def _make_sc_kernels(nch, nch_deg):
    mesh = plsc.VectorSubcoreMesh(core_axis_name="c", subcore_axis_name="s")
    deg = pl.kernel(
        functools.partial(_deg_body, nch_deg),
        out_type=_f32((_NC, _N_DEG)),
        mesh=mesh,
        scratch_types=[
            pltpu.VMEM((nch_deg, _CHUNK), jnp.int32),
            pltpu.VMEM((_CHUNK,), jnp.float32),
            pltpu.VMEM((_RPT_DEG,), jnp.float32),
            pltpu.VMEM_SHARED((_N_DEG,), jnp.float32),
        ],
    )
    agg = pl.kernel(
        functools.partial(_agg_body, nch),
        out_type=_f32((_NC, _N_ACC, _C)),
        mesh=mesh,
        scratch_types=(
            [pltpu.VMEM((_CHUNK,), jnp.int32)] * 6
            + [pltpu.VMEM((_CHUNK, _C), jnp.float32)] * 3
            + [pltpu.VMEM_SHARED((_N_ACC, _C), jnp.float32)]
            + [pltpu.SemaphoreType.DMA] * 12
        ),
    )
    return deg, agg


def _bn(x, gamma, beta):
    mean = jnp.mean(x, axis=0, keepdims=True)
    var = jnp.mean((x - mean) ** 2, axis=0, keepdims=True)
    return (x - mean) * lax.rsqrt(var + _EPS) * gamma + beta


def _dot(a, b):
    return jnp.dot(a, b, preferred_element_type=jnp.float32)


def _tcbn_body(x, g, b, xb_o):
    xb_o[...] = _bn(x[...], g[...], b[...])


def _tcz_body(d0, d1, xb, z_o, dinv_o):
    deg = d0[...] + d1[...]
    dinv = jnp.where(deg > 0.0, lax.rsqrt(jnp.maximum(deg, 1.0)), 0.0)
    z_o[...] = xb[...] * dinv
    dinv_o[...] = dinv


def _tca_body(ps, dinv, q_o, z_o):
    dv = dinv[...]
    q = dv * (ps[0] + ps[1])
    q_o[...] = q
    z_o[...] = -dv * q


def _tcb_body(xin, q, w0, w1, w2, m_o):
    m_o[...] = _dot(xin[...], w0[...] - w2[...]) - _dot(q[...], w1[...])


def _tcc_body(ps, dinv, m1, w2, b1, g2, be2, hb_o, z_o):
    dv = dinv[...]
    txw = -2.0 * dv * (ps[0, :_N] + ps[1, :_N])
    h = jax.nn.relu(m1[...] + _dot(txw, w2[...]) + b1[...])
    hb = _bn(h, g2[...], be2[...])
    hb_o[...] = hb
    z_o[...] = dv * hb


def _tcf_body(ps, dinv, xb, m2, w2, b2, y_o):
    dv = dinv[...]
    txw = -2.0 * dv * (ps[0] + ps[1])
    y_o[...] = jax.nn.relu(xb[...] + m2[...] + _dot(txw, w2[...]) + b2[...])


_G = 10
_BR = _N // _G

_ps_spec = pl.BlockSpec((_NC, _BR, _C), lambda i: (0, i, 0))
_row_spec = pl.BlockSpec((_BR, _C), lambda i: (i, 0))
_col_spec = pl.BlockSpec((_BR, 1), lambda i: (i, 0))
_w_spec = pl.BlockSpec((_C, _C), lambda i: (0, 0))
_b_spec = pl.BlockSpec((1, _C), lambda i: (0, 0))

_tcbn = pl.pallas_call(_tcbn_body, out_shape=_f32((_N, _C)))
_tcz = pl.pallas_call(
    _tcz_body, grid=(_G,),
    in_specs=[_col_spec, _col_spec, _row_spec],
    out_specs=[_row_spec, _col_spec],
    out_shape=[_f32((_N, _C)), _f32((_N, 1))])
_tca = pl.pallas_call(
    _tca_body, grid=(_G,),
    in_specs=[_ps_spec, _col_spec],
    out_specs=[_row_spec, _row_spec],
    out_shape=[_f32((_N, _C)), _f32((_N, _C))])
_tcb = pl.pallas_call(
    _tcb_body, grid=(_G,),
    in_specs=[_row_spec, _row_spec, _w_spec, _w_spec, _w_spec],
    out_specs=_row_spec,
    out_shape=_f32((_N, _C)))
_tcc = pl.pallas_call(_tcc_body, out_shape=[_f32((_N, _C)), _f32((_N, _C))])
_tcf = pl.pallas_call(
    _tcf_body, grid=(_G,),
    in_specs=[_ps_spec, _col_spec, _row_spec, _row_spec, _w_spec, _b_spec],
    out_specs=_row_spec,
    out_shape=_f32((_N, _C)))


def kernel(x, edge_index, gamma1, beta1, W1, b1, gamma2, beta2, W2, b2):
    e = edge_index.shape[1]
    ept = -(-e // _NW)
    nch = -(-ept // _CHUNK)
    while nch % 3 != 1:
        nch += 1
    nch_deg = -(-ept // _CHUNK)
    nch_deg += (-nch_deg) % 8
    src = edge_index[0]
    dst = edge_index[1]
    pad = _NW * nch * _CHUNK - e
    if pad:
        ar = jnp.arange(pad, dtype=jnp.int32)
        srcp = jnp.concatenate([src, ar % _N])
        dstp = jnp.concatenate([dst, _N + ar % (_N_ACC - _N)])
    else:
        srcp, dstp = src, dst
    pad_deg = _NW * nch_deg * _CHUNK - e
    if pad_deg:
        ar = jnp.arange(pad_deg, dtype=jnp.int32)
        dstp_deg = jnp.concatenate([dst, _N + ar % (_N_DEG - _N)])
    else:
        dstp_deg = dst
    dstp_deg = dstp_deg.reshape(_NW, nch_deg, _CHUNK)

    deg_k, agg_k = _make_sc_kernels(nch, nch_deg)

    degp = deg_k(dstp_deg)
    xb = _tcbn(x, gamma1[None], beta1[None])
    z1, dinv = _tcz(degp[0, :_N, None], degp[1, :_N, None], xb)
    ps = agg_k(z1, srcp, dstp)
    q1d, z2 = _tca(ps, dinv)
    m1 = _tcb(xb, q1d, W1[0], W1[1], W1[2])
    ps = agg_k(z2, srcp, dstp)
    hb, z3 = _tcc(ps, dinv, m1, W1[2], b1[None], gamma2[None], beta2[None])
    ps = agg_k(z3, srcp, dstp)
    q3d, z4 = _tca(ps, dinv)
    m2 = _tcb(hb, q3d, W2[0], W2[1], W2[2])
    ps = agg_k(z4, srcp, dstp)
    return _tcf(ps, dinv, xb, m2, W2[2], b2[None])
